# Initial kernel scaffold; baseline (speedup 1.0000x reference)
import jax
import jax.numpy as jnp
from jax import lax
from jax.experimental import pallas as pl
from jax.experimental.pallas import tpu as pltpu
from jax.experimental.pallas import tpu_sc as plsc

N = 10000
EDGES = 320000
F_IN = 128
H = 256
HH = 128
G = 64
C_OUT = 10

BLK = 1000
NBLK = N // BLK

NC = 2
NS = 16
CH = 128
NCHUNK = 157
EPT = NCHUNK * CH
E_PAD = EPT * NS
NP = 10240
TRASH = N
ZROWS = NP // NS


def _embed_body(x_ref, w_ref, b_ref, h0_ref, h1_ref):
    h = jnp.dot(x_ref[...], w_ref[...], preferred_element_type=jnp.float32)
    h = h + b_ref[...]
    h = jnp.where(h > 0, h, 0.01 * h)
    h0_ref[...] = h[:, :HH]
    h1_ref[...] = h[:, HH:]


def _embed(x, w, b):
    return pl.pallas_call(
        _embed_body,
        grid=(NBLK,),
        in_specs=[
            pl.BlockSpec((BLK, F_IN), lambda i: (i, 0)),
            pl.BlockSpec((F_IN, H), lambda i: (0, 0)),
            pl.BlockSpec((1, H), lambda i: (0, 0)),
        ],
        out_specs=[
            pl.BlockSpec((BLK, HH), lambda i: (i, 0)),
            pl.BlockSpec((BLK, HH), lambda i: (i, 0)),
        ],
        out_shape=[jax.ShapeDtypeStruct((N, HH), jnp.float32)] * 2,
    )(x, w, b.reshape(1, H))


def _agg_body(h0, h1, srcm, dstm, zeros_hbm, out0, out1,
              idx_src, idx_dst, rows, acc, sem):
    c = lax.axis_index("c")
    s = lax.axis_index("s")
    pltpu.sync_copy(zeros_hbm, acc.at[pl.ds(s * ZROWS, ZROWS)])
    pltpu.sync_copy(srcm.at[s], idx_src)
    pltpu.sync_copy(dstm.at[s], idx_dst)
    plsc.subcore_barrier()

    def body(i, carry):
        @pl.when(c == 0)
        def _():
            pltpu.async_copy(h0.at[idx_src.at[i]], rows, sem).wait()

        @pl.when(c == 1)
        def _():
            pltpu.async_copy(h1.at[idx_src.at[i]], rows, sem).wait()

        pltpu.sync_copy(rows, acc.at[idx_dst.at[i]], add=True)
        return carry

    lax.fori_loop(0, NCHUNK, body, 0)
    plsc.subcore_barrier()

    rpt = N // NS
    @pl.when(c == 0)
    def _():
        pltpu.sync_copy(acc.at[pl.ds(s * rpt, rpt)], out0.at[pl.ds(s * rpt, rpt)])

    @pl.when(c == 1)
    def _():
        pltpu.sync_copy(acc.at[pl.ds(s * rpt, rpt)], out1.at[pl.ds(s * rpt, rpt)])


_agg_call = pl.kernel(
    _agg_body,
    out_type=[jax.ShapeDtypeStruct((N, HH), jnp.float32)] * 2,
    mesh=plsc.VectorSubcoreMesh(
        core_axis_name="c", subcore_axis_name="s",
        num_cores=NC, num_subcores=NS),
    scratch_types=[
        pltpu.VMEM((NCHUNK, CH), jnp.int32),
        pltpu.VMEM((NCHUNK, CH), jnp.int32),
        pltpu.VMEM((CH, HH), jnp.float32),
        pltpu.VMEM_SHARED((NP, HH), jnp.float32),
        pltpu.SemaphoreType.DMA,
    ],
)


def _convlin_body(a0, a1, h0, h1, wrel, wroot, b, c0, c1, ssum, ssq,
                  accs, accq):
    i = pl.program_id(0)
    a = jnp.concatenate([a0[...], a1[...]], axis=1)
    hh = jnp.concatenate([h0[...], h1[...]], axis=1)
    out = (jnp.dot(a, wrel[...], preferred_element_type=jnp.float32)
           + jnp.dot(hh, wroot[...], preferred_element_type=jnp.float32)
           + b[...])
    c0[...] = out[:, :HH]
    c1[...] = out[:, HH:]

    @pl.when(i == 0)
    def _():
        accs[...] = jnp.zeros_like(accs)
        accq[...] = jnp.zeros_like(accq)

    accs[...] += jnp.sum(out, axis=0, keepdims=True)
    accq[...] += jnp.sum(out * out, axis=0, keepdims=True)

    @pl.when(i == NBLK - 1)
    def _():
        ssum[...] = accs[...]
        ssq[...] = accq[...]


def _convlin(a0, a1, h0, h1, wrel, wroot, b):
    return pl.pallas_call(
        _convlin_body,
        grid=(NBLK,),
        in_specs=[
            pl.BlockSpec((BLK, HH), lambda i: (i, 0)),
            pl.BlockSpec((BLK, HH), lambda i: (i, 0)),
            pl.BlockSpec((BLK, HH), lambda i: (i, 0)),
            pl.BlockSpec((BLK, HH), lambda i: (i, 0)),
            pl.BlockSpec((H, H), lambda i: (0, 0)),
            pl.BlockSpec((H, H), lambda i: (0, 0)),
            pl.BlockSpec((1, H), lambda i: (0, 0)),
        ],
        out_specs=[
            pl.BlockSpec((BLK, HH), lambda i: (i, 0)),
            pl.BlockSpec((BLK, HH), lambda i: (i, 0)),
            pl.BlockSpec((1, H), lambda i: (0, 0)),
            pl.BlockSpec((1, H), lambda i: (0, 0)),
        ],
        out_shape=[
            jax.ShapeDtypeStruct((N, HH), jnp.float32),
            jax.ShapeDtypeStruct((N, HH), jnp.float32),
            jax.ShapeDtypeStruct((1, H), jnp.float32),
            jax.ShapeDtypeStruct((1, H), jnp.float32),
        ],
        scratch_shapes=[
            pltpu.VMEM((1, H), jnp.float32),
            pltpu.VMEM((1, H), jnp.float32),
        ],
    )(a0, a1, h0, h1, wrel, wroot, b.reshape(1, H))


def _bnapply_body(c0, c1, ssum, ssq, gm, bt, i0, i1, o0, o1):
    mean = ssum[...] * (1.0 / N)
    var = ssq[...] * (1.0 / N) - mean * mean
    scale = gm[...] * lax.rsqrt(var + 1e-5)
    v = jnp.concatenate([c0[...], c1[...]], axis=1)
    y = (v - mean) * scale + bt[...]
    y = jnp.where(y > 0, y, 0.01 * y)
    y = y + jnp.concatenate([i0[...], i1[...]], axis=1)
    o0[...] = y[:, :HH]
    o1[...] = y[:, HH:]


def _bnapply(c0, c1, ssum, ssq, gm, bt, i0, i1):
    return pl.pallas_call(
        _bnapply_body,
        grid=(NBLK,),
        in_specs=[
            pl.BlockSpec((BLK, HH), lambda i: (i, 0)),
            pl.BlockSpec((BLK, HH), lambda i: (i, 0)),
            pl.BlockSpec((1, H), lambda i: (0, 0)),
            pl.BlockSpec((1, H), lambda i: (0, 0)),
            pl.BlockSpec((1, H), lambda i: (0, 0)),
            pl.BlockSpec((1, H), lambda i: (0, 0)),
            pl.BlockSpec((BLK, HH), lambda i: (i, 0)),
            pl.BlockSpec((BLK, HH), lambda i: (i, 0)),
        ],
        out_specs=[
            pl.BlockSpec((BLK, HH), lambda i: (i, 0)),
            pl.BlockSpec((BLK, HH), lambda i: (i, 0)),
        ],
        out_shape=[jax.ShapeDtypeStruct((N, HH), jnp.float32)] * 2,
    )(c0, c1, ssum, ssq, gm.reshape(1, H), bt.reshape(1, H), i0, i1)


def _pool_body(h0, h1, bt3, wf1, bf1, wf2, bf2, wf3, bf3, out, accp, accc):
    i = pl.program_id(0)

    @pl.when(i == 0)
    def _():
        accp[...] = jnp.zeros_like(accp)
        accc[...] = jnp.zeros_like(accc)

    bvals = bt3[0, 0, :]
    oh = (bvals[:, None] == lax.broadcasted_iota(jnp.int32, (BLK, G), 1)
          ).astype(jnp.float32)
    hh = jnp.concatenate([h0[...], h1[...]], axis=1)
    accp[...] += lax.dot_general(oh, hh, (((0,), (0,)), ((), ())),
                                 preferred_element_type=jnp.float32)
    accc[...] += jnp.sum(oh, axis=0).reshape(G, 1)

    @pl.when(i == NBLK - 1)
    def _():
        pooled = accp[...] / jnp.maximum(accc[...], 1.0)
        f1 = jnp.dot(pooled, wf1[...], preferred_element_type=jnp.float32) + bf1[...]
        f1 = jnp.where(f1 > 0, f1, 0.01 * f1)
        f2 = jnp.dot(f1, wf2[...], preferred_element_type=jnp.float32) + bf2[...]
        f2 = jnp.where(f2 > 0, f2, 0.01 * f2)
        out[...] = jnp.dot(f2, wf3[...], preferred_element_type=jnp.float32) + bf3[...]


def _pool_mlp(h0, h1, batch, wf1, bf1, wf2, bf2, wf3, bf3):
    bt3 = batch.astype(jnp.int32).reshape(NBLK, 1, BLK)
    return pl.pallas_call(
        _pool_body,
        grid=(NBLK,),
        in_specs=[
            pl.BlockSpec((BLK, HH), lambda i: (i, 0)),
            pl.BlockSpec((BLK, HH), lambda i: (i, 0)),
            pl.BlockSpec((1, 1, BLK), lambda i: (i, 0, 0)),
            pl.BlockSpec((H, H), lambda i: (0, 0)),
            pl.BlockSpec((1, H), lambda i: (0, 0)),
            pl.BlockSpec((H, H // 2), lambda i: (0, 0)),
            pl.BlockSpec((1, H // 2), lambda i: (0, 0)),
            pl.BlockSpec((H // 2, C_OUT), lambda i: (0, 0)),
            pl.BlockSpec((1, C_OUT), lambda i: (0, 0)),
        ],
        out_specs=pl.BlockSpec((G, C_OUT), lambda i: (0, 0)),
        out_shape=jax.ShapeDtypeStruct((G, C_OUT), jnp.float32),
        scratch_shapes=[
            pltpu.VMEM((G, H), jnp.float32),
            pltpu.VMEM((G, 1), jnp.float32),
        ],
    )(h0, h1, bt3, wf1, bf1.reshape(1, H), wf2, bf2.reshape(1, H // 2),
      wf3, bf3.reshape(1, C_OUT))


def kernel(x, edge_index, batch, W_embed, b_embed, W1_rel, b1_rel, W1_root,
           W2_rel, b2_rel, W2_root, bn1_g, bn1_b, bn2_g, bn2_b,
           Wf1, bf1, Wf2, bf2, Wf3, bf3):
    src = edge_index[0].astype(jnp.int32)
    dst = edge_index[1].astype(jnp.int32)
    pad = E_PAD - EDGES
    srcm = jnp.concatenate([src, jnp.zeros((pad,), jnp.int32)]
                           ).reshape(NS, NCHUNK, CH)
    dstm = jnp.concatenate([dst, jnp.full((pad,), TRASH, jnp.int32)]
                           ).reshape(NS, NCHUNK, CH)
    zeros_in = jnp.zeros((ZROWS, HH), jnp.float32)

    h0, h1 = _embed(x, W_embed, b_embed)
    a0, a1 = _agg_call(h0, h1, srcm, dstm, zeros_in)
    c0, c1, s1, q1 = _convlin(a0, a1, h0, h1, W1_rel, W1_root, b1_rel)
    g0, g1 = _bnapply(c0, c1, s1, q1, bn1_g, bn1_b, h0, h1)
    a0, a1 = _agg_call(g0, g1, srcm, dstm, zeros_in)
    c0, c1, s2, q2 = _convlin(a0, a1, g0, g1, W2_rel, W2_root, b2_rel)
    g0, g1 = _bnapply(c0, c1, s2, q2, bn2_g, bn2_b, h0, h1)
    return _pool_mlp(g0, g1, batch, Wf1, bf1, Wf2, bf2, Wf3, bf3)


# speedup vs baseline: 2.5799x; 2.5799x over previous
import jax
import jax.numpy as jnp
from jax import lax
from jax.experimental import pallas as pl
from jax.experimental.pallas import tpu as pltpu
from jax.experimental.pallas import tpu_sc as plsc

N = 10000
EDGES = 320000
F_IN = 128
H = 256
HH = 128
G = 64
C_OUT = 10

BLK = 1000
NBLK = N // BLK

NC = 2
NS = 16
CH = 128
NCHUNK = 160
GRP = 8
NGRP = NCHUNK // GRP
EPT = NCHUNK * CH
E_PAD = EPT * NS
NP = 10240
TRASH = N
ZROWS = NP // NS


def _embed_body(x_ref, w_ref, b_ref, h0_ref, h1_ref):
    h = jnp.dot(x_ref[...], w_ref[...], preferred_element_type=jnp.float32)
    h = h + b_ref[...]
    h = jnp.where(h > 0, h, 0.01 * h)
    h0_ref[...] = h[:, :HH]
    h1_ref[...] = h[:, HH:]


def _embed(x, w, b):
    return pl.pallas_call(
        _embed_body,
        grid=(NBLK,),
        in_specs=[
            pl.BlockSpec((BLK, F_IN), lambda i: (i, 0)),
            pl.BlockSpec((F_IN, H), lambda i: (0, 0)),
            pl.BlockSpec((1, H), lambda i: (0, 0)),
        ],
        out_specs=[
            pl.BlockSpec((BLK, HH), lambda i: (i, 0)),
            pl.BlockSpec((BLK, HH), lambda i: (i, 0)),
        ],
        out_shape=[jax.ShapeDtypeStruct((N, HH), jnp.float32)] * 2,
    )(x, w, b.reshape(1, H))


def _agg_body(h0, h1, srcm, dstm, zeros_hbm, out0, out1,
              idx_src, idx_dst, rows, acc, sem):
    c = lax.axis_index("c")
    s = lax.axis_index("s")
    pltpu.sync_copy(zeros_hbm, acc.at[pl.ds(s * ZROWS, ZROWS)])
    plsc.subcore_barrier()

    def group(g, carry):
        pltpu.sync_copy(srcm.at[s].at[pl.ds(g * GRP, GRP)], idx_src)
        pltpu.sync_copy(dstm.at[s].at[pl.ds(g * GRP, GRP)], idx_dst)

        def body(j, carry2):
            @pl.when(c == 0)
            def _():
                pltpu.async_copy(h0.at[idx_src.at[j]], rows, sem).wait()

            @pl.when(c == 1)
            def _():
                pltpu.async_copy(h1.at[idx_src.at[j]], rows, sem).wait()

            pltpu.sync_copy(rows, acc.at[idx_dst.at[j]], add=True)
            return carry2

        return lax.fori_loop(0, GRP, body, carry)

    lax.fori_loop(0, NGRP, group, 0)
    plsc.subcore_barrier()

    @pl.when(c == 0)
    def _():
        pltpu.sync_copy(acc.at[pl.ds(s * ZROWS, ZROWS)],
                        out0.at[pl.ds(s * ZROWS, ZROWS)])

    @pl.when(c == 1)
    def _():
        pltpu.sync_copy(acc.at[pl.ds(s * ZROWS, ZROWS)],
                        out1.at[pl.ds(s * ZROWS, ZROWS)])


import functools


@functools.cache
def _make_agg_call():
    return pl.kernel(
        _agg_body,
        out_type=[jax.ShapeDtypeStruct((NP, HH), jnp.float32)] * 2,
        mesh=plsc.VectorSubcoreMesh(
            core_axis_name="c", subcore_axis_name="s",
            num_cores=NC, num_subcores=NS),
        scratch_types=[
            pltpu.VMEM((GRP, CH), jnp.int32),
            pltpu.VMEM((GRP, CH), jnp.int32),
            pltpu.VMEM((CH, HH), jnp.float32),
            pltpu.VMEM_SHARED((NP, HH), jnp.float32),
            pltpu.SemaphoreType.DMA,
        ],
    )


def _convlin_body(a0, a1, h0, h1, wrel, wroot, b, c0, c1, ssum, ssq,
                  accs, accq):
    i = pl.program_id(0)
    a = jnp.concatenate([a0[...], a1[...]], axis=1)
    hh = jnp.concatenate([h0[...], h1[...]], axis=1)
    out = (jnp.dot(a, wrel[...], preferred_element_type=jnp.float32)
           + jnp.dot(hh, wroot[...], preferred_element_type=jnp.float32)
           + b[...])
    c0[...] = out[:, :HH]
    c1[...] = out[:, HH:]

    @pl.when(i == 0)
    def _():
        accs[...] = jnp.zeros_like(accs)
        accq[...] = jnp.zeros_like(accq)

    accs[...] += jnp.sum(out, axis=0, keepdims=True)
    accq[...] += jnp.sum(out * out, axis=0, keepdims=True)

    @pl.when(i == NBLK - 1)
    def _():
        ssum[...] = accs[...]
        ssq[...] = accq[...]


def _convlin(a0, a1, h0, h1, wrel, wroot, b):
    return pl.pallas_call(
        _convlin_body,
        grid=(NBLK,),
        in_specs=[
            pl.BlockSpec((BLK, HH), lambda i: (i, 0)),
            pl.BlockSpec((BLK, HH), lambda i: (i, 0)),
            pl.BlockSpec((BLK, HH), lambda i: (i, 0)),
            pl.BlockSpec((BLK, HH), lambda i: (i, 0)),
            pl.BlockSpec((H, H), lambda i: (0, 0)),
            pl.BlockSpec((H, H), lambda i: (0, 0)),
            pl.BlockSpec((1, H), lambda i: (0, 0)),
        ],
        out_specs=[
            pl.BlockSpec((BLK, HH), lambda i: (i, 0)),
            pl.BlockSpec((BLK, HH), lambda i: (i, 0)),
            pl.BlockSpec((1, H), lambda i: (0, 0)),
            pl.BlockSpec((1, H), lambda i: (0, 0)),
        ],
        out_shape=[
            jax.ShapeDtypeStruct((N, HH), jnp.float32),
            jax.ShapeDtypeStruct((N, HH), jnp.float32),
            jax.ShapeDtypeStruct((1, H), jnp.float32),
            jax.ShapeDtypeStruct((1, H), jnp.float32),
        ],
        scratch_shapes=[
            pltpu.VMEM((1, H), jnp.float32),
            pltpu.VMEM((1, H), jnp.float32),
        ],
    )(a0, a1, h0, h1, wrel, wroot, b.reshape(1, H))


def _bnapply_body(c0, c1, ssum, ssq, gm, bt, i0, i1, o0, o1):
    mean = ssum[...] * (1.0 / N)
    var = ssq[...] * (1.0 / N) - mean * mean
    scale = gm[...] * lax.rsqrt(var + 1e-5)
    v = jnp.concatenate([c0[...], c1[...]], axis=1)
    y = (v - mean) * scale + bt[...]
    y = jnp.where(y > 0, y, 0.01 * y)
    y = y + jnp.concatenate([i0[...], i1[...]], axis=1)
    o0[...] = y[:, :HH]
    o1[...] = y[:, HH:]


def _bnapply(c0, c1, ssum, ssq, gm, bt, i0, i1):
    return pl.pallas_call(
        _bnapply_body,
        grid=(NBLK,),
        in_specs=[
            pl.BlockSpec((BLK, HH), lambda i: (i, 0)),
            pl.BlockSpec((BLK, HH), lambda i: (i, 0)),
            pl.BlockSpec((1, H), lambda i: (0, 0)),
            pl.BlockSpec((1, H), lambda i: (0, 0)),
            pl.BlockSpec((1, H), lambda i: (0, 0)),
            pl.BlockSpec((1, H), lambda i: (0, 0)),
            pl.BlockSpec((BLK, HH), lambda i: (i, 0)),
            pl.BlockSpec((BLK, HH), lambda i: (i, 0)),
        ],
        out_specs=[
            pl.BlockSpec((BLK, HH), lambda i: (i, 0)),
            pl.BlockSpec((BLK, HH), lambda i: (i, 0)),
        ],
        out_shape=[jax.ShapeDtypeStruct((N, HH), jnp.float32)] * 2,
    )(c0, c1, ssum, ssq, gm.reshape(1, H), bt.reshape(1, H), i0, i1)


def _pool_body(h0, h1, bt3, wf1, bf1, wf2, bf2, wf3, bf3, out, accp, accc):
    i = pl.program_id(0)

    @pl.when(i == 0)
    def _():
        accp[...] = jnp.zeros_like(accp)
        accc[...] = jnp.zeros_like(accc)

    bvals = bt3[0, 0, :]
    oh = (bvals[:, None] == lax.broadcasted_iota(jnp.int32, (BLK, G), 1)
          ).astype(jnp.float32)
    hh = jnp.concatenate([h0[...], h1[...]], axis=1)
    accp[...] += lax.dot_general(oh, hh, (((0,), (0,)), ((), ())),
                                 preferred_element_type=jnp.float32)
    accc[...] += jnp.sum(oh, axis=0).reshape(G, 1)

    @pl.when(i == NBLK - 1)
    def _():
        pooled = accp[...] / jnp.maximum(accc[...], 1.0)
        f1 = jnp.dot(pooled, wf1[...], preferred_element_type=jnp.float32) + bf1[...]
        f1 = jnp.where(f1 > 0, f1, 0.01 * f1)
        f2 = jnp.dot(f1, wf2[...], preferred_element_type=jnp.float32) + bf2[...]
        f2 = jnp.where(f2 > 0, f2, 0.01 * f2)
        out[...] = jnp.dot(f2, wf3[...], preferred_element_type=jnp.float32) + bf3[...]


def _pool_mlp(h0, h1, batch, wf1, bf1, wf2, bf2, wf3, bf3):
    bt3 = batch.astype(jnp.int32).reshape(NBLK, 1, BLK)
    return pl.pallas_call(
        _pool_body,
        grid=(NBLK,),
        in_specs=[
            pl.BlockSpec((BLK, HH), lambda i: (i, 0)),
            pl.BlockSpec((BLK, HH), lambda i: (i, 0)),
            pl.BlockSpec((1, 1, BLK), lambda i: (i, 0, 0)),
            pl.BlockSpec((H, H), lambda i: (0, 0)),
            pl.BlockSpec((1, H), lambda i: (0, 0)),
            pl.BlockSpec((H, H // 2), lambda i: (0, 0)),
            pl.BlockSpec((1, H // 2), lambda i: (0, 0)),
            pl.BlockSpec((H // 2, C_OUT), lambda i: (0, 0)),
            pl.BlockSpec((1, C_OUT), lambda i: (0, 0)),
        ],
        out_specs=pl.BlockSpec((G, C_OUT), lambda i: (0, 0)),
        out_shape=jax.ShapeDtypeStruct((G, C_OUT), jnp.float32),
        scratch_shapes=[
            pltpu.VMEM((G, H), jnp.float32),
            pltpu.VMEM((G, 1), jnp.float32),
        ],
    )(h0, h1, bt3, wf1, bf1.reshape(1, H), wf2, bf2.reshape(1, H // 2),
      wf3, bf3.reshape(1, C_OUT))


def kernel(x, edge_index, batch, W_embed, b_embed, W1_rel, b1_rel, W1_root,
           W2_rel, b2_rel, W2_root, bn1_g, bn1_b, bn2_g, bn2_b,
           Wf1, bf1, Wf2, bf2, Wf3, bf3):
    src = edge_index[0].astype(jnp.int32)
    dst = edge_index[1].astype(jnp.int32)
    pad = E_PAD - EDGES
    srcm = jnp.concatenate([src, jnp.zeros((pad,), jnp.int32)]
                           ).reshape(NS, NCHUNK, CH)
    dstm = jnp.concatenate([dst, jnp.full((pad,), TRASH, jnp.int32)]
                           ).reshape(NS, NCHUNK, CH)
    zeros_in = jnp.zeros((ZROWS, HH), jnp.float32)

    agg = _make_agg_call()
    h0, h1 = _embed(x, W_embed, b_embed)
    a0, a1 = agg(h0, h1, srcm, dstm, zeros_in)
    a0, a1 = a0[:N], a1[:N]
    c0, c1, s1, q1 = _convlin(a0, a1, h0, h1, W1_rel, W1_root, b1_rel)
    g0, g1 = _bnapply(c0, c1, s1, q1, bn1_g, bn1_b, h0, h1)
    a0, a1 = agg(g0, g1, srcm, dstm, zeros_in)
    a0, a1 = a0[:N], a1[:N]
    c0, c1, s2, q2 = _convlin(a0, a1, g0, g1, W2_rel, W2_root, b2_rel)
    g0, g1 = _bnapply(c0, c1, s2, q2, bn2_g, bn2_b, h0, h1)
    return _pool_mlp(g0, g1, batch, Wf1, bf1, Wf2, bf2, Wf3, bf3)


# speedup vs baseline: 2.9928x; 1.1600x over previous
import jax
import jax.numpy as jnp
from jax import lax
from jax.experimental import pallas as pl
from jax.experimental.pallas import tpu as pltpu
from jax.experimental.pallas import tpu_sc as plsc

N = 10000
EDGES = 320000
F_IN = 128
H = 256
HH = 128
G = 64
C_OUT = 10

BLK = 1000
NBLK = N // BLK

NC = 2
NS = 16
CH = 64
NCHUNK = 320
GRP = 8
NGRP = NCHUNK // GRP
EPT = NCHUNK * CH
E_PAD = EPT * NS
NP = 10240
TRASH = N
ZROWS = NP // NS


def _embed_body(x_ref, w_ref, b_ref, h0_ref, h1_ref):
    h = jnp.dot(x_ref[...], w_ref[...], preferred_element_type=jnp.float32)
    h = h + b_ref[...]
    h = jnp.where(h > 0, h, 0.01 * h)
    h0_ref[...] = h[:, :HH]
    h1_ref[...] = h[:, HH:]


def _embed(x, w, b):
    return pl.pallas_call(
        _embed_body,
        grid=(NBLK,),
        in_specs=[
            pl.BlockSpec((BLK, F_IN), lambda i: (i, 0)),
            pl.BlockSpec((F_IN, H), lambda i: (0, 0)),
            pl.BlockSpec((1, H), lambda i: (0, 0)),
        ],
        out_specs=[
            pl.BlockSpec((BLK, HH), lambda i: (i, 0)),
            pl.BlockSpec((BLK, HH), lambda i: (i, 0)),
        ],
        out_shape=[jax.ShapeDtypeStruct((N, HH), jnp.float32)] * 2,
    )(x, w, b.reshape(1, H))


def _agg_body(h0, h1, srcm, dstm, zeros_hbm, out0, out1,
              idx_src, idx_dst, rows0, rows1, acc, sem0, sem1):
    c = lax.axis_index("c")
    s = lax.axis_index("s")
    pltpu.sync_copy(zeros_hbm, acc.at[pl.ds(s * ZROWS, ZROWS)])
    plsc.subcore_barrier()

    rbufs = (rows0, rows1)
    sems = (sem0, sem1)

    def issue(j, b):
        @pl.when(c == 0)
        def _():
            pltpu.async_copy(h0.at[idx_src.at[j]], rbufs[b], sems[b])

        @pl.when(c == 1)
        def _():
            pltpu.async_copy(h1.at[idx_src.at[j]], rbufs[b], sems[b])

    def wait(b):
        pltpu.make_async_copy(h0.at[idx_src.at[0]], rbufs[b], sems[b]).wait()

    def group(g, carry):
        pltpu.sync_copy(srcm.at[s].at[pl.ds(g * GRP, GRP)], idx_src)
        pltpu.sync_copy(dstm.at[s].at[pl.ds(g * GRP, GRP)], idx_dst)
        issue(0, 0)
        for j in range(GRP):
            if j + 1 < GRP:
                issue(j + 1, (j + 1) % 2)
            wait(j % 2)
            pltpu.sync_copy(rbufs[j % 2], acc.at[idx_dst.at[j]], add=True)
        return carry

    lax.fori_loop(0, NGRP, group, 0)
    plsc.subcore_barrier()

    @pl.when(c == 0)
    def _():
        pltpu.sync_copy(acc.at[pl.ds(s * ZROWS, ZROWS)],
                        out0.at[pl.ds(s * ZROWS, ZROWS)])

    @pl.when(c == 1)
    def _():
        pltpu.sync_copy(acc.at[pl.ds(s * ZROWS, ZROWS)],
                        out1.at[pl.ds(s * ZROWS, ZROWS)])


import functools


@functools.cache
def _make_agg_call():
    return pl.kernel(
        _agg_body,
        out_type=[jax.ShapeDtypeStruct((NP, HH), jnp.float32)] * 2,
        mesh=plsc.VectorSubcoreMesh(
            core_axis_name="c", subcore_axis_name="s",
            num_cores=NC, num_subcores=NS),
        scratch_types=[
            pltpu.VMEM((GRP, CH), jnp.int32),
            pltpu.VMEM((GRP, CH), jnp.int32),
            pltpu.VMEM((CH, HH), jnp.float32),
            pltpu.VMEM((CH, HH), jnp.float32),
            pltpu.VMEM_SHARED((NP, HH), jnp.float32),
            pltpu.SemaphoreType.DMA,
            pltpu.SemaphoreType.DMA,
        ],
    )


def _convlin_body(a0, a1, h0, h1, wrel, wroot, b, c0, c1, ssum, ssq,
                  accs, accq):
    i = pl.program_id(0)
    a = jnp.concatenate([a0[...], a1[...]], axis=1)
    hh = jnp.concatenate([h0[...], h1[...]], axis=1)
    out = (jnp.dot(a, wrel[...], preferred_element_type=jnp.float32)
           + jnp.dot(hh, wroot[...], preferred_element_type=jnp.float32)
           + b[...])
    c0[...] = out[:, :HH]
    c1[...] = out[:, HH:]

    @pl.when(i == 0)
    def _():
        accs[...] = jnp.zeros_like(accs)
        accq[...] = jnp.zeros_like(accq)

    accs[...] += jnp.sum(out, axis=0, keepdims=True)
    accq[...] += jnp.sum(out * out, axis=0, keepdims=True)

    @pl.when(i == NBLK - 1)
    def _():
        ssum[...] = accs[...]
        ssq[...] = accq[...]


def _convlin(a0, a1, h0, h1, wrel, wroot, b):
    return pl.pallas_call(
        _convlin_body,
        grid=(NBLK,),
        in_specs=[
            pl.BlockSpec((BLK, HH), lambda i: (i, 0)),
            pl.BlockSpec((BLK, HH), lambda i: (i, 0)),
            pl.BlockSpec((BLK, HH), lambda i: (i, 0)),
            pl.BlockSpec((BLK, HH), lambda i: (i, 0)),
            pl.BlockSpec((H, H), lambda i: (0, 0)),
            pl.BlockSpec((H, H), lambda i: (0, 0)),
            pl.BlockSpec((1, H), lambda i: (0, 0)),
        ],
        out_specs=[
            pl.BlockSpec((BLK, HH), lambda i: (i, 0)),
            pl.BlockSpec((BLK, HH), lambda i: (i, 0)),
            pl.BlockSpec((1, H), lambda i: (0, 0)),
            pl.BlockSpec((1, H), lambda i: (0, 0)),
        ],
        out_shape=[
            jax.ShapeDtypeStruct((N, HH), jnp.float32),
            jax.ShapeDtypeStruct((N, HH), jnp.float32),
            jax.ShapeDtypeStruct((1, H), jnp.float32),
            jax.ShapeDtypeStruct((1, H), jnp.float32),
        ],
        scratch_shapes=[
            pltpu.VMEM((1, H), jnp.float32),
            pltpu.VMEM((1, H), jnp.float32),
        ],
    )(a0, a1, h0, h1, wrel, wroot, b.reshape(1, H))


def _bnapply_body(c0, c1, ssum, ssq, gm, bt, i0, i1, o0, o1):
    mean = ssum[...] * (1.0 / N)
    var = ssq[...] * (1.0 / N) - mean * mean
    scale = gm[...] * lax.rsqrt(var + 1e-5)
    v = jnp.concatenate([c0[...], c1[...]], axis=1)
    y = (v - mean) * scale + bt[...]
    y = jnp.where(y > 0, y, 0.01 * y)
    y = y + jnp.concatenate([i0[...], i1[...]], axis=1)
    o0[...] = y[:, :HH]
    o1[...] = y[:, HH:]


def _bnapply(c0, c1, ssum, ssq, gm, bt, i0, i1):
    return pl.pallas_call(
        _bnapply_body,
        grid=(NBLK,),
        in_specs=[
            pl.BlockSpec((BLK, HH), lambda i: (i, 0)),
            pl.BlockSpec((BLK, HH), lambda i: (i, 0)),
            pl.BlockSpec((1, H), lambda i: (0, 0)),
            pl.BlockSpec((1, H), lambda i: (0, 0)),
            pl.BlockSpec((1, H), lambda i: (0, 0)),
            pl.BlockSpec((1, H), lambda i: (0, 0)),
            pl.BlockSpec((BLK, HH), lambda i: (i, 0)),
            pl.BlockSpec((BLK, HH), lambda i: (i, 0)),
        ],
        out_specs=[
            pl.BlockSpec((BLK, HH), lambda i: (i, 0)),
            pl.BlockSpec((BLK, HH), lambda i: (i, 0)),
        ],
        out_shape=[jax.ShapeDtypeStruct((N, HH), jnp.float32)] * 2,
    )(c0, c1, ssum, ssq, gm.reshape(1, H), bt.reshape(1, H), i0, i1)


def _pool_body(h0, h1, bt3, wf1, bf1, wf2, bf2, wf3, bf3, out, accp, accc):
    i = pl.program_id(0)

    @pl.when(i == 0)
    def _():
        accp[...] = jnp.zeros_like(accp)
        accc[...] = jnp.zeros_like(accc)

    bvals = bt3[0, 0, :]
    oh = (bvals[:, None] == lax.broadcasted_iota(jnp.int32, (BLK, G), 1)
          ).astype(jnp.float32)
    hh = jnp.concatenate([h0[...], h1[...]], axis=1)
    accp[...] += lax.dot_general(oh, hh, (((0,), (0,)), ((), ())),
                                 preferred_element_type=jnp.float32)
    accc[...] += jnp.sum(oh, axis=0).reshape(G, 1)

    @pl.when(i == NBLK - 1)
    def _():
        pooled = accp[...] / jnp.maximum(accc[...], 1.0)
        f1 = jnp.dot(pooled, wf1[...], preferred_element_type=jnp.float32) + bf1[...]
        f1 = jnp.where(f1 > 0, f1, 0.01 * f1)
        f2 = jnp.dot(f1, wf2[...], preferred_element_type=jnp.float32) + bf2[...]
        f2 = jnp.where(f2 > 0, f2, 0.01 * f2)
        out[...] = jnp.dot(f2, wf3[...], preferred_element_type=jnp.float32) + bf3[...]


def _pool_mlp(h0, h1, batch, wf1, bf1, wf2, bf2, wf3, bf3):
    bt3 = batch.astype(jnp.int32).reshape(NBLK, 1, BLK)
    return pl.pallas_call(
        _pool_body,
        grid=(NBLK,),
        in_specs=[
            pl.BlockSpec((BLK, HH), lambda i: (i, 0)),
            pl.BlockSpec((BLK, HH), lambda i: (i, 0)),
            pl.BlockSpec((1, 1, BLK), lambda i: (i, 0, 0)),
            pl.BlockSpec((H, H), lambda i: (0, 0)),
            pl.BlockSpec((1, H), lambda i: (0, 0)),
            pl.BlockSpec((H, H // 2), lambda i: (0, 0)),
            pl.BlockSpec((1, H // 2), lambda i: (0, 0)),
            pl.BlockSpec((H // 2, C_OUT), lambda i: (0, 0)),
            pl.BlockSpec((1, C_OUT), lambda i: (0, 0)),
        ],
        out_specs=pl.BlockSpec((G, C_OUT), lambda i: (0, 0)),
        out_shape=jax.ShapeDtypeStruct((G, C_OUT), jnp.float32),
        scratch_shapes=[
            pltpu.VMEM((G, H), jnp.float32),
            pltpu.VMEM((G, 1), jnp.float32),
        ],
    )(h0, h1, bt3, wf1, bf1.reshape(1, H), wf2, bf2.reshape(1, H // 2),
      wf3, bf3.reshape(1, C_OUT))


def kernel(x, edge_index, batch, W_embed, b_embed, W1_rel, b1_rel, W1_root,
           W2_rel, b2_rel, W2_root, bn1_g, bn1_b, bn2_g, bn2_b,
           Wf1, bf1, Wf2, bf2, Wf3, bf3):
    src = edge_index[0].astype(jnp.int32)
    dst = edge_index[1].astype(jnp.int32)
    pad = E_PAD - EDGES
    srcm = jnp.concatenate([src, jnp.zeros((pad,), jnp.int32)]
                           ).reshape(NS, NCHUNK, CH)
    dstm = jnp.concatenate([dst, jnp.full((pad,), TRASH, jnp.int32)]
                           ).reshape(NS, NCHUNK, CH)
    zeros_in = jnp.zeros((ZROWS, HH), jnp.float32)

    agg = _make_agg_call()
    h0, h1 = _embed(x, W_embed, b_embed)
    a0, a1 = agg(h0, h1, srcm, dstm, zeros_in)
    a0, a1 = a0[:N], a1[:N]
    c0, c1, s1, q1 = _convlin(a0, a1, h0, h1, W1_rel, W1_root, b1_rel)
    g0, g1 = _bnapply(c0, c1, s1, q1, bn1_g, bn1_b, h0, h1)
    a0, a1 = agg(g0, g1, srcm, dstm, zeros_in)
    a0, a1 = a0[:N], a1[:N]
    c0, c1, s2, q2 = _convlin(a0, a1, g0, g1, W2_rel, W2_root, b2_rel)
    g0, g1 = _bnapply(c0, c1, s2, q2, bn2_g, bn2_b, h0, h1)
    return _pool_mlp(g0, g1, batch, Wf1, bf1, Wf2, bf2, Wf3, bf3)


# speedup vs baseline: 4.2141x; 1.4081x over previous
import functools

import jax
import jax.numpy as jnp
from jax import lax
from jax.experimental import pallas as pl
from jax.experimental.pallas import tpu as pltpu
from jax.experimental.pallas import tpu_sc as plsc

N = 10000
EDGES = 320000
F_IN = 128
H = 256
G = 64
C_OUT = 10

BLK = 1000
NBLK = N // BLK

NC = 2
NS = 16
SPLIT = 5120
ACC_R = 5248
AROWS = ACC_R // NS
PAD_ROW = 5120
CH = 64
CAP = 10240
EB = EDGES // (NC * NS)
NVREG = EB // 16


def _bin_body(srce, dste, src_lists, dst_lists, counts,
              sv, dv, lo_s, lo_d, hi_s, hi_d, cvec):
    c = lax.axis_index("c")
    s = lax.axis_index("s")
    w = c * NS + s
    pltpu.sync_copy(srce.at[w], sv)
    pltpu.sync_copy(dste.at[w], dv)

    def vloop(i, carry):
        lo, hi = carry
        iot = lax.iota(jnp.int32, 16)
        svv = sv[pl.ds(i * 16, 16)]
        dvv = dv[pl.ds(i * 16, 16)]
        m_lo = dvv < SPLIT
        m_hi = jnp.logical_not(m_lo)
        dn = lax.GatherDimensionNumbers(
            offset_dims=(), collapsed_slice_dims=(0,), start_index_map=(0,))
        x = jnp.where(m_lo, 1, 0)
        for k in (1, 2, 4, 8):
            g = lax.gather(x, jnp.maximum(iot - k, 0)[:, None], dn,
                           slice_sizes=(1,),
                           mode=lax.GatherScatterMode.PROMISE_IN_BOUNDS)
            x = x + jnp.where(iot >= k, g, 0)
        incl_lo = x
        excl_lo = incl_lo - jnp.where(m_lo, 1, 0)
        off_lo = jnp.full((16,), lo, jnp.int32) + excl_lo
        off_hi = jnp.full((16,), hi, jnp.int32) + iot - excl_lo
        plsc.store_scatter(lo_s, [off_lo], svv, mask=m_lo)
        plsc.store_scatter(lo_d, [off_lo], dvv, mask=m_lo)
        plsc.store_scatter(hi_s, [off_hi], svv, mask=m_hi)
        plsc.store_scatter(hi_d, [off_hi], dvv - SPLIT, mask=m_hi)
        nlo = lax.squeeze(lax.slice(incl_lo, (15,), (16,)), (0,))
        return (lo + nlo, hi + (16 - nlo))

    lo, hi = lax.fori_loop(0, NVREG, vloop, (0, 0))

    pad_s = jnp.zeros((16,), jnp.int32)
    pad_d = jnp.full((16,), PAD_ROW, jnp.int32)
    for k in range(9):
        lo_s[pl.ds(lo + k * 16, 16)] = pad_s
        lo_d[pl.ds(lo + k * 16, 16)] = pad_d
        hi_s[pl.ds(hi + k * 16, 16)] = pad_s
        hi_d[pl.ds(hi + k * 16, 16)] = pad_d

    nch_lo = 2 * ((lo + 127) // 128)
    nch_hi = 2 * ((hi + 127) // 128)
    iota = lax.iota(jnp.int32, 16)
    cvec[...] = (jnp.where(iota == 0, jnp.full((16,), nch_lo, jnp.int32), 0)
                 + jnp.where(iota == 1, jnp.full((16,), nch_hi, jnp.int32), 0))
    pltpu.sync_copy(cvec, counts.at[w])
    pltpu.sync_copy(lo_s, src_lists.at[0, w])
    pltpu.sync_copy(lo_d, dst_lists.at[0, w])
    pltpu.sync_copy(hi_s, src_lists.at[1, w])
    pltpu.sync_copy(hi_d, dst_lists.at[1, w])


@functools.cache
def _make_bin_call():
    return pl.kernel(
        _bin_body,
        out_type=[
            jax.ShapeDtypeStruct((NC, NC * NS, CAP), jnp.int32),
            jax.ShapeDtypeStruct((NC, NC * NS, CAP), jnp.int32),
            jax.ShapeDtypeStruct((NC * NS, 16), jnp.int32),
        ],
        mesh=plsc.VectorSubcoreMesh(
            core_axis_name="c", subcore_axis_name="s",
            num_cores=NC, num_subcores=NS),
        compiler_params=pltpu.CompilerParams(needs_layout_passes=False),
        scratch_types=[
            pltpu.VMEM((EB,), jnp.int32),
            pltpu.VMEM((EB,), jnp.int32),
            pltpu.VMEM((CAP,), jnp.int32),
            pltpu.VMEM((CAP,), jnp.int32),
            pltpu.VMEM((CAP,), jnp.int32),
            pltpu.VMEM((CAP,), jnp.int32),
            pltpu.VMEM((16,), jnp.int32),
        ],
    )


def _agg_body(hfull, src_lists, dst_lists, counts, zeros_hbm, out,
              is0, id0, is1, id1, rows0, rows1, cnt_v, acc, sem0, sem1):
    c = lax.axis_index("c")
    s = lax.axis_index("s")
    pltpu.sync_copy(zeros_hbm, acc.at[pl.ds(s * AROWS, AROWS)])
    pltpu.sync_copy(counts, cnt_v)
    plsc.subcore_barrier()

    for o in range(2):
        w = 2 * s + o
        crow = cnt_v[pl.ds(w * 16, 16)]
        n0 = lax.squeeze(lax.slice(crow, (0,), (1,)), (0,))
        n1 = lax.squeeze(lax.slice(crow, (1,), (2,)), (0,))
        nch = jnp.where(c == 0, n0, n1)
        npair = nch // 2
        slist = src_lists.at[c, w]
        dlist = dst_lists.at[c, w]

        def pair(p, carry):
            e0 = p * 2 * CH
            e1 = e0 + CH
            pltpu.sync_copy(slist.at[pl.ds(e0, CH)], is0.at[0])
            pltpu.sync_copy(dlist.at[pl.ds(e0, CH)], id0.at[0])
            pltpu.async_copy(hfull.at[is0.at[0]], rows0, sem0)
            pltpu.sync_copy(slist.at[pl.ds(e1, CH)], is1.at[0])
            pltpu.sync_copy(dlist.at[pl.ds(e1, CH)], id1.at[0])
            pltpu.async_copy(hfull.at[is1.at[0]], rows1, sem1)
            pltpu.make_async_copy(hfull.at[is0.at[0]], rows0, sem0).wait()
            pltpu.sync_copy(rows0, acc.at[id0.at[0]], add=True)
            pltpu.make_async_copy(hfull.at[is1.at[0]], rows1, sem1).wait()
            pltpu.sync_copy(rows1, acc.at[id1.at[0]], add=True)
            return carry

        lax.fori_loop(0, npair, pair, 0)

    plsc.subcore_barrier()
    pltpu.sync_copy(acc.at[pl.ds(s * AROWS, AROWS)],
                    out.at[pl.ds(c * ACC_R + s * AROWS, AROWS)])


@functools.cache
def _make_agg_call():
    return pl.kernel(
        _agg_body,
        out_type=jax.ShapeDtypeStruct((NC * ACC_R, 2, 128), jnp.float32),
        mesh=plsc.VectorSubcoreMesh(
            core_axis_name="c", subcore_axis_name="s",
            num_cores=NC, num_subcores=NS),
        scratch_types=[
            pltpu.VMEM((1, CH), jnp.int32),
            pltpu.VMEM((1, CH), jnp.int32),
            pltpu.VMEM((1, CH), jnp.int32),
            pltpu.VMEM((1, CH), jnp.int32),
            pltpu.VMEM((CH, 2, 128), jnp.float32),
            pltpu.VMEM((CH, 2, 128), jnp.float32),
            pltpu.VMEM((NC * NS * 16,), jnp.int32),
            pltpu.VMEM_SHARED((ACC_R, 2, 128), jnp.float32),
            pltpu.SemaphoreType.DMA,
            pltpu.SemaphoreType.DMA,
        ],
    )


def _embed_body(x_ref, w_ref, b_ref, h_ref):
    h = jnp.dot(x_ref[...], w_ref[...], preferred_element_type=jnp.float32)
    h = h + b_ref[...]
    h_ref[...] = jnp.where(h > 0, h, 0.01 * h)


def _embed(x, w, b):
    return pl.pallas_call(
        _embed_body,
        grid=(NBLK,),
        in_specs=[
            pl.BlockSpec((BLK, F_IN), lambda i: (i, 0)),
            pl.BlockSpec((F_IN, H), lambda i: (0, 0)),
            pl.BlockSpec((1, H), lambda i: (0, 0)),
        ],
        out_specs=pl.BlockSpec((BLK, H), lambda i: (i, 0)),
        out_shape=jax.ShapeDtypeStruct((N, H), jnp.float32),
    )(x, w, b.reshape(1, H))


def _convlin_body(a_ref, h_ref, wrel, wroot, b, c_ref, ssum, ssq, accs, accq):
    i = pl.program_id(0)
    out = (jnp.dot(a_ref[...], wrel[...], preferred_element_type=jnp.float32)
           + jnp.dot(h_ref[...], wroot[...], preferred_element_type=jnp.float32)
           + b[...])
    c_ref[...] = out

    @pl.when(i == 0)
    def _():
        accs[...] = jnp.zeros_like(accs)
        accq[...] = jnp.zeros_like(accq)

    accs[...] += jnp.sum(out, axis=0, keepdims=True)
    accq[...] += jnp.sum(out * out, axis=0, keepdims=True)

    @pl.when(i == NBLK - 1)
    def _():
        ssum[...] = accs[...]
        ssq[...] = accq[...]


def _convlin(a, h, wrel, wroot, b):
    return pl.pallas_call(
        _convlin_body,
        grid=(NBLK,),
        in_specs=[
            pl.BlockSpec((BLK, H), lambda i: (i, 0)),
            pl.BlockSpec((BLK, H), lambda i: (i, 0)),
            pl.BlockSpec((H, H), lambda i: (0, 0)),
            pl.BlockSpec((H, H), lambda i: (0, 0)),
            pl.BlockSpec((1, H), lambda i: (0, 0)),
        ],
        out_specs=[
            pl.BlockSpec((BLK, H), lambda i: (i, 0)),
            pl.BlockSpec((1, H), lambda i: (0, 0)),
            pl.BlockSpec((1, H), lambda i: (0, 0)),
        ],
        out_shape=[
            jax.ShapeDtypeStruct((N, H), jnp.float32),
            jax.ShapeDtypeStruct((1, H), jnp.float32),
            jax.ShapeDtypeStruct((1, H), jnp.float32),
        ],
        scratch_shapes=[
            pltpu.VMEM((1, H), jnp.float32),
            pltpu.VMEM((1, H), jnp.float32),
        ],
    )(a, h, wrel, wroot, b.reshape(1, H))


def _bnapply_body(c_ref, ssum, ssq, gm, bt, id_ref, o_ref):
    mean = ssum[...] * (1.0 / N)
    var = ssq[...] * (1.0 / N) - mean * mean
    scale = gm[...] * lax.rsqrt(var + 1e-5)
    y = (c_ref[...] - mean) * scale + bt[...]
    y = jnp.where(y > 0, y, 0.01 * y)
    o_ref[...] = y + id_ref[...]


def _bnapply(cc, ssum, ssq, gm, bt, iden):
    return pl.pallas_call(
        _bnapply_body,
        grid=(NBLK,),
        in_specs=[
            pl.BlockSpec((BLK, H), lambda i: (i, 0)),
            pl.BlockSpec((1, H), lambda i: (0, 0)),
            pl.BlockSpec((1, H), lambda i: (0, 0)),
            pl.BlockSpec((1, H), lambda i: (0, 0)),
            pl.BlockSpec((1, H), lambda i: (0, 0)),
            pl.BlockSpec((BLK, H), lambda i: (i, 0)),
        ],
        out_specs=pl.BlockSpec((BLK, H), lambda i: (i, 0)),
        out_shape=jax.ShapeDtypeStruct((N, H), jnp.float32),
    )(cc, ssum, ssq, gm.reshape(1, H), bt.reshape(1, H), iden)


def _pool_body(h_ref, bt3, wf1, bf1, wf2, bf2, wf3, bf3, out, accp, accc):
    i = pl.program_id(0)

    @pl.when(i == 0)
    def _():
        accp[...] = jnp.zeros_like(accp)
        accc[...] = jnp.zeros_like(accc)

    bvals = bt3[0, 0, :]
    oh = (bvals[:, None] == lax.broadcasted_iota(jnp.int32, (BLK, G), 1)
          ).astype(jnp.float32)
    accp[...] += lax.dot_general(oh, h_ref[...], (((0,), (0,)), ((), ())),
                                 preferred_element_type=jnp.float32)
    accc[...] += jnp.sum(oh, axis=0).reshape(G, 1)

    @pl.when(i == NBLK - 1)
    def _():
        pooled = accp[...] / jnp.maximum(accc[...], 1.0)
        f1 = jnp.dot(pooled, wf1[...], preferred_element_type=jnp.float32) + bf1[...]
        f1 = jnp.where(f1 > 0, f1, 0.01 * f1)
        f2 = jnp.dot(f1, wf2[...], preferred_element_type=jnp.float32) + bf2[...]
        f2 = jnp.where(f2 > 0, f2, 0.01 * f2)
        out[...] = jnp.dot(f2, wf3[...], preferred_element_type=jnp.float32) + bf3[...]


def _pool_mlp(h, batch, wf1, bf1, wf2, bf2, wf3, bf3):
    bt3 = batch.astype(jnp.int32).reshape(NBLK, 1, BLK)
    return pl.pallas_call(
        _pool_body,
        grid=(NBLK,),
        in_specs=[
            pl.BlockSpec((BLK, H), lambda i: (i, 0)),
            pl.BlockSpec((1, 1, BLK), lambda i: (i, 0, 0)),
            pl.BlockSpec((H, H), lambda i: (0, 0)),
            pl.BlockSpec((1, H), lambda i: (0, 0)),
            pl.BlockSpec((H, H // 2), lambda i: (0, 0)),
            pl.BlockSpec((1, H // 2), lambda i: (0, 0)),
            pl.BlockSpec((H // 2, C_OUT), lambda i: (0, 0)),
            pl.BlockSpec((1, C_OUT), lambda i: (0, 0)),
        ],
        out_specs=pl.BlockSpec((G, C_OUT), lambda i: (0, 0)),
        out_shape=jax.ShapeDtypeStruct((G, C_OUT), jnp.float32),
        scratch_shapes=[
            pltpu.VMEM((G, H), jnp.float32),
            pltpu.VMEM((G, 1), jnp.float32),
        ],
    )(h, bt3, wf1, bf1.reshape(1, H), wf2, bf2.reshape(1, H // 2),
      wf3, bf3.reshape(1, C_OUT))


def kernel(x, edge_index, batch, W_embed, b_embed, W1_rel, b1_rel, W1_root,
           W2_rel, b2_rel, W2_root, bn1_g, bn1_b, bn2_g, bn2_b,
           Wf1, bf1, Wf2, bf2, Wf3, bf3):
    srce = edge_index[0].astype(jnp.int32).reshape(NC * NS, EB)
    dste = edge_index[1].astype(jnp.int32).reshape(NC * NS, EB)
    zeros_in = jnp.zeros((AROWS, 2, 128), jnp.float32)

    src_lists, dst_lists, counts = _make_bin_call()(srce, dste)
    counts_flat = counts.reshape(-1)
    agg = _make_agg_call()

    def conv_agg(hf):
        o = agg(hf.reshape(N, 2, 128), src_lists, dst_lists, counts_flat,
                zeros_in)
        o = o.reshape(NC * ACC_R, H)
        return jnp.concatenate([o[:SPLIT], o[ACC_R:ACC_R + N - SPLIT]], axis=0)

    hf = _embed(x, W_embed, b_embed)
    a = conv_agg(hf)
    cc, s1, q1 = _convlin(a, hf, W1_rel, W1_root, b1_rel)
    g1 = _bnapply(cc, s1, q1, bn1_g, bn1_b, hf)
    a = conv_agg(g1)
    cc, s2, q2 = _convlin(a, g1, W2_rel, W2_root, b2_rel)
    g2 = _bnapply(cc, s2, q2, bn2_g, bn2_b, hf)
    return _pool_mlp(g2, batch, Wf1, bf1, Wf2, bf2, Wf3, bf3)


# speedup vs baseline: 4.4744x; 1.0618x over previous
import functools

import jax
import jax.numpy as jnp
from jax import lax
from jax.experimental import pallas as pl
from jax.experimental.pallas import tpu as pltpu
from jax.experimental.pallas import tpu_sc as plsc

N = 10000
EDGES = 320000
F_IN = 128
H = 256
G = 64
C_OUT = 10

BLK = 1000
NBLK = N // BLK

NC = 2
NS = 16
SPLIT = 5120
ACC_R = 5248
AROWS = ACC_R // NS
PAD_ROW = 5120
CH = 64
CAP = 10240
EB = EDGES // (NC * NS)
NVREG = EB // 16


def _bin_body(srce, dste, comb, counts,
              sv, dv, lo_b, hi_b, cvec):
    c = lax.axis_index("c")
    s = lax.axis_index("s")
    w = c * NS + s
    pltpu.sync_copy(srce.at[w], sv)
    pltpu.sync_copy(dste.at[w], dv)

    def _flat(e):
        return ((e >> 6) << 7) + (e & 63)

    def vloop(i, carry):
        lo, hi = carry
        iot = lax.iota(jnp.int32, 16)
        svv = sv[pl.ds(i * 16, 16)]
        dvv = dv[pl.ds(i * 16, 16)]
        m_lo = dvv < SPLIT
        m_hi = jnp.logical_not(m_lo)
        dn = lax.GatherDimensionNumbers(
            offset_dims=(), collapsed_slice_dims=(0,), start_index_map=(0,))
        x = jnp.where(m_lo, 1, 0)
        for k in (1, 2, 4, 8):
            g = lax.gather(x, jnp.maximum(iot - k, 0)[:, None], dn,
                           slice_sizes=(1,),
                           mode=lax.GatherScatterMode.PROMISE_IN_BOUNDS)
            x = x + jnp.where(iot >= k, g, 0)
        incl_lo = x
        excl_lo = incl_lo - jnp.where(m_lo, 1, 0)
        f_lo = _flat(jnp.full((16,), lo, jnp.int32) + excl_lo)
        f_hi = _flat(jnp.full((16,), hi, jnp.int32) + iot - excl_lo)
        plsc.store_scatter(lo_b, [f_lo], svv, mask=m_lo)
        plsc.store_scatter(lo_b, [f_lo + 64], dvv, mask=m_lo)
        plsc.store_scatter(hi_b, [f_hi], svv, mask=m_hi)
        plsc.store_scatter(hi_b, [f_hi + 64], dvv - SPLIT, mask=m_hi)
        nlo = lax.squeeze(lax.slice(incl_lo, (15,), (16,)), (0,))
        return (lo + nlo, hi + (16 - nlo))

    lo, hi = lax.fori_loop(0, NVREG, vloop, (0, 0))

    iota = lax.iota(jnp.int32, 16)
    pad_s = jnp.zeros((16,), jnp.int32)
    pad_d = jnp.full((16,), PAD_ROW, jnp.int32)
    for k in range(9):
        e_lo = _flat(jnp.full((16,), lo + k * 16, jnp.int32) + iota)
        e_hi = _flat(jnp.full((16,), hi + k * 16, jnp.int32) + iota)
        plsc.store_scatter(lo_b, [e_lo], pad_s, mask=e_lo == e_lo)
        plsc.store_scatter(lo_b, [e_lo + 64], pad_d, mask=e_lo == e_lo)
        plsc.store_scatter(hi_b, [e_hi], pad_s, mask=e_hi == e_hi)
        plsc.store_scatter(hi_b, [e_hi + 64], pad_d, mask=e_hi == e_hi)

    np_lo = (lo + 127) // 128
    np_hi = (hi + 127) // 128
    cvec[...] = (jnp.where(iota == 0, jnp.full((16,), np_lo, jnp.int32), 0)
                 + jnp.where(iota == 1, jnp.full((16,), np_hi, jnp.int32), 0))
    pltpu.sync_copy(cvec, counts.at[w])
    pltpu.sync_copy(lo_b, comb.at[0, w])
    pltpu.sync_copy(hi_b, comb.at[1, w])


@functools.cache
def _make_bin_call():
    return pl.kernel(
        _bin_body,
        out_type=[
            jax.ShapeDtypeStruct((NC, NC * NS, 2 * CAP), jnp.int32),
            jax.ShapeDtypeStruct((NC * NS, 16), jnp.int32),
        ],
        mesh=plsc.VectorSubcoreMesh(
            core_axis_name="c", subcore_axis_name="s",
            num_cores=NC, num_subcores=NS),
        compiler_params=pltpu.CompilerParams(needs_layout_passes=False),
        scratch_types=[
            pltpu.VMEM((EB,), jnp.int32),
            pltpu.VMEM((EB,), jnp.int32),
            pltpu.VMEM((2 * CAP,), jnp.int32),
            pltpu.VMEM((2 * CAP,), jnp.int32),
            pltpu.VMEM((16,), jnp.int32),
        ],
    )


def _agg_body(hfull, comb, counts, zeros_hbm, out,
              ibuf0, ibuf1, rows0, rows1, cnt_v, acc, sem0, sem1,
              ssem0, ssem1):
    c = lax.axis_index("c")
    s = lax.axis_index("s")
    pltpu.sync_copy(zeros_hbm, acc.at[pl.ds(s * AROWS, AROWS)])
    pltpu.sync_copy(counts, cnt_v)
    plsc.subcore_barrier()

    for o in range(2):
        w = 2 * s + o
        crow = cnt_v[pl.ds(w * 16, 16)]
        n0 = lax.squeeze(lax.slice(crow, (0,), (1,)), (0,))
        n1 = lax.squeeze(lax.slice(crow, (1,), (2,)), (0,))
        npair = jnp.where(c == 0, n0, n1)
        clist = comb.at[c, w]
        ibufs = (ibuf0, ibuf1)

        def quad(q, carry):
            for par in range(2):
                p = 2 * q + par
                ib = ibufs[par]

                @pl.when(p < npair)
                def _():
                    pltpu.sync_copy(clist.at[p], ib)

                    @pl.when(p > 0)
                    def _():
                        pltpu.make_async_copy(rows0, acc.at[ib.at[1]],
                                              ssem0).wait()
                        pltpu.make_async_copy(rows1, acc.at[ib.at[3]],
                                              ssem1).wait()

                    pltpu.async_copy(hfull.at[ib.at[0]], rows0, sem0)
                    pltpu.async_copy(hfull.at[ib.at[2]], rows1, sem1)
                    pltpu.make_async_copy(hfull.at[ib.at[0]], rows0,
                                          sem0).wait()
                    pltpu.async_copy(rows0, acc.at[ib.at[1]], ssem0, add=True)
                    pltpu.make_async_copy(hfull.at[ib.at[2]], rows1,
                                          sem1).wait()
                    pltpu.async_copy(rows1, acc.at[ib.at[3]], ssem1, add=True)
            return carry

        lax.fori_loop(0, (npair + 1) // 2, quad, 0)

        @pl.when(npair > 0)
        def _():
            pltpu.make_async_copy(rows0, acc.at[ibuf0.at[1]], ssem0).wait()
            pltpu.make_async_copy(rows1, acc.at[ibuf0.at[3]], ssem1).wait()

    plsc.subcore_barrier()
    pltpu.sync_copy(acc.at[pl.ds(s * AROWS, AROWS)],
                    out.at[pl.ds(c * ACC_R + s * AROWS, AROWS)])


@functools.cache
def _make_agg_call():
    return pl.kernel(
        _agg_body,
        out_type=jax.ShapeDtypeStruct((NC * ACC_R, 2, 128), jnp.float32),
        mesh=plsc.VectorSubcoreMesh(
            core_axis_name="c", subcore_axis_name="s",
            num_cores=NC, num_subcores=NS),
        scratch_types=[
            pltpu.VMEM((4, CH), jnp.int32),
            pltpu.VMEM((4, CH), jnp.int32),
            pltpu.VMEM((CH, 2, 128), jnp.float32),
            pltpu.VMEM((CH, 2, 128), jnp.float32),
            pltpu.VMEM((NC * NS * 16,), jnp.int32),
            pltpu.VMEM_SHARED((ACC_R, 2, 128), jnp.float32),
            pltpu.SemaphoreType.DMA,
            pltpu.SemaphoreType.DMA,
            pltpu.SemaphoreType.DMA,
            pltpu.SemaphoreType.DMA,
        ],
    )


def _embed_body(x_ref, w_ref, b_ref, h_ref):
    h = jnp.dot(x_ref[...], w_ref[...], preferred_element_type=jnp.float32)
    h = h + b_ref[...]
    h_ref[...] = jnp.where(h > 0, h, 0.01 * h)


def _embed(x, w, b):
    return pl.pallas_call(
        _embed_body,
        grid=(NBLK,),
        in_specs=[
            pl.BlockSpec((BLK, F_IN), lambda i: (i, 0)),
            pl.BlockSpec((F_IN, H), lambda i: (0, 0)),
            pl.BlockSpec((1, H), lambda i: (0, 0)),
        ],
        out_specs=pl.BlockSpec((BLK, H), lambda i: (i, 0)),
        out_shape=jax.ShapeDtypeStruct((N, H), jnp.float32),
    )(x, w, b.reshape(1, H))


def _convlin_body(a_ref, h_ref, wrel, wroot, b, c_ref, ssum, ssq, accs, accq):
    i = pl.program_id(0)
    out = (jnp.dot(a_ref[...], wrel[...], preferred_element_type=jnp.float32)
           + jnp.dot(h_ref[...], wroot[...], preferred_element_type=jnp.float32)
           + b[...])
    c_ref[...] = out

    @pl.when(i == 0)
    def _():
        accs[...] = jnp.zeros_like(accs)
        accq[...] = jnp.zeros_like(accq)

    accs[...] += jnp.sum(out, axis=0, keepdims=True)
    accq[...] += jnp.sum(out * out, axis=0, keepdims=True)

    @pl.when(i == NBLK - 1)
    def _():
        ssum[...] = accs[...]
        ssq[...] = accq[...]


def _convlin(a, h, wrel, wroot, b):
    return pl.pallas_call(
        _convlin_body,
        grid=(NBLK,),
        in_specs=[
            pl.BlockSpec((BLK, H), lambda i: (i, 0)),
            pl.BlockSpec((BLK, H), lambda i: (i, 0)),
            pl.BlockSpec((H, H), lambda i: (0, 0)),
            pl.BlockSpec((H, H), lambda i: (0, 0)),
            pl.BlockSpec((1, H), lambda i: (0, 0)),
        ],
        out_specs=[
            pl.BlockSpec((BLK, H), lambda i: (i, 0)),
            pl.BlockSpec((1, H), lambda i: (0, 0)),
            pl.BlockSpec((1, H), lambda i: (0, 0)),
        ],
        out_shape=[
            jax.ShapeDtypeStruct((N, H), jnp.float32),
            jax.ShapeDtypeStruct((1, H), jnp.float32),
            jax.ShapeDtypeStruct((1, H), jnp.float32),
        ],
        scratch_shapes=[
            pltpu.VMEM((1, H), jnp.float32),
            pltpu.VMEM((1, H), jnp.float32),
        ],
    )(a, h, wrel, wroot, b.reshape(1, H))


def _bnapply_body(c_ref, ssum, ssq, gm, bt, id_ref, o_ref):
    mean = ssum[...] * (1.0 / N)
    var = ssq[...] * (1.0 / N) - mean * mean
    scale = gm[...] * lax.rsqrt(var + 1e-5)
    y = (c_ref[...] - mean) * scale + bt[...]
    y = jnp.where(y > 0, y, 0.01 * y)
    o_ref[...] = y + id_ref[...]


def _bnapply(cc, ssum, ssq, gm, bt, iden):
    return pl.pallas_call(
        _bnapply_body,
        grid=(NBLK,),
        in_specs=[
            pl.BlockSpec((BLK, H), lambda i: (i, 0)),
            pl.BlockSpec((1, H), lambda i: (0, 0)),
            pl.BlockSpec((1, H), lambda i: (0, 0)),
            pl.BlockSpec((1, H), lambda i: (0, 0)),
            pl.BlockSpec((1, H), lambda i: (0, 0)),
            pl.BlockSpec((BLK, H), lambda i: (i, 0)),
        ],
        out_specs=pl.BlockSpec((BLK, H), lambda i: (i, 0)),
        out_shape=jax.ShapeDtypeStruct((N, H), jnp.float32),
    )(cc, ssum, ssq, gm.reshape(1, H), bt.reshape(1, H), iden)


def _pool_body(h_ref, bt3, wf1, bf1, wf2, bf2, wf3, bf3, out, accp, accc):
    i = pl.program_id(0)

    @pl.when(i == 0)
    def _():
        accp[...] = jnp.zeros_like(accp)
        accc[...] = jnp.zeros_like(accc)

    bvals = bt3[0, 0, :]
    oh = (bvals[:, None] == lax.broadcasted_iota(jnp.int32, (BLK, G), 1)
          ).astype(jnp.float32)
    accp[...] += lax.dot_general(oh, h_ref[...], (((0,), (0,)), ((), ())),
                                 preferred_element_type=jnp.float32)
    accc[...] += jnp.sum(oh, axis=0).reshape(G, 1)

    @pl.when(i == NBLK - 1)
    def _():
        pooled = accp[...] / jnp.maximum(accc[...], 1.0)
        f1 = jnp.dot(pooled, wf1[...], preferred_element_type=jnp.float32) + bf1[...]
        f1 = jnp.where(f1 > 0, f1, 0.01 * f1)
        f2 = jnp.dot(f1, wf2[...], preferred_element_type=jnp.float32) + bf2[...]
        f2 = jnp.where(f2 > 0, f2, 0.01 * f2)
        out[...] = jnp.dot(f2, wf3[...], preferred_element_type=jnp.float32) + bf3[...]


def _pool_mlp(h, batch, wf1, bf1, wf2, bf2, wf3, bf3):
    bt3 = batch.astype(jnp.int32).reshape(NBLK, 1, BLK)
    return pl.pallas_call(
        _pool_body,
        grid=(NBLK,),
        in_specs=[
            pl.BlockSpec((BLK, H), lambda i: (i, 0)),
            pl.BlockSpec((1, 1, BLK), lambda i: (i, 0, 0)),
            pl.BlockSpec((H, H), lambda i: (0, 0)),
            pl.BlockSpec((1, H), lambda i: (0, 0)),
            pl.BlockSpec((H, H // 2), lambda i: (0, 0)),
            pl.BlockSpec((1, H // 2), lambda i: (0, 0)),
            pl.BlockSpec((H // 2, C_OUT), lambda i: (0, 0)),
            pl.BlockSpec((1, C_OUT), lambda i: (0, 0)),
        ],
        out_specs=pl.BlockSpec((G, C_OUT), lambda i: (0, 0)),
        out_shape=jax.ShapeDtypeStruct((G, C_OUT), jnp.float32),
        scratch_shapes=[
            pltpu.VMEM((G, H), jnp.float32),
            pltpu.VMEM((G, 1), jnp.float32),
        ],
    )(h, bt3, wf1, bf1.reshape(1, H), wf2, bf2.reshape(1, H // 2),
      wf3, bf3.reshape(1, C_OUT))


def kernel(x, edge_index, batch, W_embed, b_embed, W1_rel, b1_rel, W1_root,
           W2_rel, b2_rel, W2_root, bn1_g, bn1_b, bn2_g, bn2_b,
           Wf1, bf1, Wf2, bf2, Wf3, bf3):
    srce = edge_index[0].astype(jnp.int32).reshape(NC * NS, EB)
    dste = edge_index[1].astype(jnp.int32).reshape(NC * NS, EB)
    zeros_in = jnp.zeros((AROWS, 2, 128), jnp.float32)

    comb, counts = _make_bin_call()(srce, dste)
    comb4 = comb.reshape(NC, NC * NS, CAP // 128, 4, CH)
    counts_flat = counts.reshape(-1)
    agg = _make_agg_call()

    def conv_agg(hf):
        o = agg(hf.reshape(N, 2, 128), comb4, counts_flat, zeros_in)
        o = o.reshape(NC * ACC_R, H)
        return jnp.concatenate([o[:SPLIT], o[ACC_R:ACC_R + N - SPLIT]], axis=0)

    hf = _embed(x, W_embed, b_embed)
    a = conv_agg(hf)
    cc, s1, q1 = _convlin(a, hf, W1_rel, W1_root, b1_rel)
    g1 = _bnapply(cc, s1, q1, bn1_g, bn1_b, hf)
    a = conv_agg(g1)
    cc, s2, q2 = _convlin(a, g1, W2_rel, W2_root, b2_rel)
    g2 = _bnapply(cc, s2, q2, bn2_g, bn2_b, hf)
    return _pool_mlp(g2, batch, Wf1, bf1, Wf2, bf2, Wf3, bf3)


# speedup vs baseline: 5.3091x; 1.1865x over previous
import functools

import jax
import jax.numpy as jnp
from jax import lax
from jax.experimental import pallas as pl
from jax.experimental.pallas import tpu as pltpu
from jax.experimental.pallas import tpu_sc as plsc

N = 10000
EDGES = 320000
F_IN = 128
H = 256
G = 64
C_OUT = 10

BLK = 1000
NBLK = N // BLK

NC = 2
NS = 16
SPLIT = 5120
ACC_R = 5248
AROWS = ACC_R // NS
PAD_ROW = 5120
CH = 32
CAP = 10240
EB = EDGES // (NC * NS)
NVREG = EB // 16


def _bin_body(srce, dste, comb, counts,
              sv, dv, lo_b, hi_b, cvec):
    c = lax.axis_index("c")
    s = lax.axis_index("s")
    w = c * NS + s
    pltpu.sync_copy(srce.at[w], sv)
    pltpu.sync_copy(dste.at[w], dv)

    def _flat(e):
        return ((e >> 5) << 6) + (e & 31)

    def vloop(i, carry):
        lo, hi = carry
        iot = lax.iota(jnp.int32, 16)
        svv = sv[pl.ds(i * 16, 16)]
        dvv = dv[pl.ds(i * 16, 16)]
        m_lo = dvv < SPLIT
        m_hi = jnp.logical_not(m_lo)
        dn = lax.GatherDimensionNumbers(
            offset_dims=(), collapsed_slice_dims=(0,), start_index_map=(0,))
        x = jnp.where(m_lo, 1, 0)
        for k in (1, 2, 4, 8):
            g = lax.gather(x, jnp.maximum(iot - k, 0)[:, None], dn,
                           slice_sizes=(1,),
                           mode=lax.GatherScatterMode.PROMISE_IN_BOUNDS)
            x = x + jnp.where(iot >= k, g, 0)
        incl_lo = x
        excl_lo = incl_lo - jnp.where(m_lo, 1, 0)
        f_lo = _flat(jnp.full((16,), lo, jnp.int32) + excl_lo)
        f_hi = _flat(jnp.full((16,), hi, jnp.int32) + iot - excl_lo)
        plsc.store_scatter(lo_b, [f_lo], svv, mask=m_lo)
        plsc.store_scatter(lo_b, [f_lo + 32], dvv, mask=m_lo)
        plsc.store_scatter(hi_b, [f_hi], svv, mask=m_hi)
        plsc.store_scatter(hi_b, [f_hi + 32], dvv - SPLIT, mask=m_hi)
        nlo = lax.squeeze(lax.slice(incl_lo, (15,), (16,)), (0,))
        return (lo + nlo, hi + (16 - nlo))

    lo, hi = lax.fori_loop(0, NVREG, vloop, (0, 0))

    iota = lax.iota(jnp.int32, 16)
    pad_s = jnp.zeros((16,), jnp.int32)
    pad_d = jnp.full((16,), PAD_ROW, jnp.int32)
    for k in range(5):
        e_lo = _flat(jnp.full((16,), lo + k * 16, jnp.int32) + iota)
        e_hi = _flat(jnp.full((16,), hi + k * 16, jnp.int32) + iota)
        plsc.store_scatter(lo_b, [e_lo], pad_s, mask=e_lo == e_lo)
        plsc.store_scatter(lo_b, [e_lo + 32], pad_d, mask=e_lo == e_lo)
        plsc.store_scatter(hi_b, [e_hi], pad_s, mask=e_hi == e_hi)
        plsc.store_scatter(hi_b, [e_hi + 32], pad_d, mask=e_hi == e_hi)

    np_lo = (lo + 63) // 64
    np_hi = (hi + 63) // 64
    cvec[...] = (jnp.where(iota == 0, jnp.full((16,), np_lo, jnp.int32), 0)
                 + jnp.where(iota == 1, jnp.full((16,), np_hi, jnp.int32), 0))
    pltpu.sync_copy(cvec, counts.at[w])
    pltpu.sync_copy(lo_b, comb.at[0, w])
    pltpu.sync_copy(hi_b, comb.at[1, w])


@functools.cache
def _make_bin_call():
    return pl.kernel(
        _bin_body,
        out_type=[
            jax.ShapeDtypeStruct((NC, NC * NS, 2 * CAP), jnp.int32),
            jax.ShapeDtypeStruct((NC * NS, 16), jnp.int32),
        ],
        mesh=plsc.VectorSubcoreMesh(
            core_axis_name="c", subcore_axis_name="s",
            num_cores=NC, num_subcores=NS),
        compiler_params=pltpu.CompilerParams(needs_layout_passes=False),
        scratch_types=[
            pltpu.VMEM((EB,), jnp.int32),
            pltpu.VMEM((EB,), jnp.int32),
            pltpu.VMEM((2 * CAP,), jnp.int32),
            pltpu.VMEM((2 * CAP,), jnp.int32),
            pltpu.VMEM((16,), jnp.int32),
        ],
    )


def _agg_body(hfull, comb, counts, zeros_hbm, out,
              ibuf0, ibuf1, rows0, rows1, rows2, rows3, cnt_v, acc,
              sem0, sem1, sem2, sem3, ssem0, ssem1, ssem2, ssem3):
    c = lax.axis_index("c")
    s = lax.axis_index("s")
    pltpu.sync_copy(zeros_hbm, acc.at[pl.ds(s * AROWS, AROWS)])
    pltpu.sync_copy(counts, cnt_v)
    plsc.subcore_barrier()

    for o in range(2):
        w = 2 * s + o
        crow = cnt_v[pl.ds(w * 16, 16)]
        n0 = lax.squeeze(lax.slice(crow, (0,), (1,)), (0,))
        n1 = lax.squeeze(lax.slice(crow, (1,), (2,)), (0,))
        npair = jnp.where(c == 0, n0, n1)
        clist = comb.at[c, w]
        ibufs = (ibuf0, ibuf1)
        rpairs = ((rows0, rows1), (rows2, rows3))
        gsems = ((sem0, sem1), (sem2, sem3))
        ssems = ((ssem0, ssem1), (ssem2, ssem3))

        def quad(q, carry):
            for par in range(2):
                p = 2 * q + par
                ib = ibufs[par]
                r0, r1 = rpairs[par]
                g0, g1 = gsems[par]
                t0, t1 = ssems[par]

                @pl.when(p < npair)
                def _():
                    @pl.when(p > 1)
                    def _():
                        pltpu.make_async_copy(r0, acc.at[ib.at[1]],
                                              t0).wait()
                        pltpu.make_async_copy(r1, acc.at[ib.at[3]],
                                              t1).wait()

                    pltpu.sync_copy(clist.at[p], ib)
                    pltpu.async_copy(hfull.at[ib.at[0]], r0, g0)
                    pltpu.async_copy(hfull.at[ib.at[2]], r1, g1)

            for par in range(2):
                p = 2 * q + par
                ib = ibufs[par]
                r0, r1 = rpairs[par]
                g0, g1 = gsems[par]
                t0, t1 = ssems[par]

                @pl.when(p < npair)
                def _():
                    pltpu.make_async_copy(hfull.at[ib.at[0]], r0, g0).wait()
                    pltpu.async_copy(r0, acc.at[ib.at[1]], t0, add=True)
                    pltpu.make_async_copy(hfull.at[ib.at[2]], r1, g1).wait()
                    pltpu.async_copy(r1, acc.at[ib.at[3]], t1, add=True)
            return carry

        lax.fori_loop(0, (npair + 1) // 2, quad, 0)

        @pl.when(npair > 0)
        def _():
            pltpu.make_async_copy(rows0, acc.at[ibuf0.at[1]], ssem0).wait()
            pltpu.make_async_copy(rows1, acc.at[ibuf0.at[3]], ssem1).wait()

        @pl.when(npair > 1)
        def _():
            pltpu.make_async_copy(rows2, acc.at[ibuf1.at[1]], ssem2).wait()
            pltpu.make_async_copy(rows3, acc.at[ibuf1.at[3]], ssem3).wait()

    plsc.subcore_barrier()
    pltpu.sync_copy(acc.at[pl.ds(s * AROWS, AROWS)],
                    out.at[pl.ds(c * ACC_R + s * AROWS, AROWS)])


@functools.cache
def _make_agg_call():
    return pl.kernel(
        _agg_body,
        out_type=jax.ShapeDtypeStruct((NC * ACC_R, 2, 128), jnp.float32),
        mesh=plsc.VectorSubcoreMesh(
            core_axis_name="c", subcore_axis_name="s",
            num_cores=NC, num_subcores=NS),
        scratch_types=[
            pltpu.VMEM((4, CH), jnp.int32),
            pltpu.VMEM((4, CH), jnp.int32),
            pltpu.VMEM((CH, 2, 128), jnp.float32),
            pltpu.VMEM((CH, 2, 128), jnp.float32),
            pltpu.VMEM((CH, 2, 128), jnp.float32),
            pltpu.VMEM((CH, 2, 128), jnp.float32),
            pltpu.VMEM((NC * NS * 16,), jnp.int32),
            pltpu.VMEM_SHARED((ACC_R, 2, 128), jnp.float32),
            pltpu.SemaphoreType.DMA,
            pltpu.SemaphoreType.DMA,
            pltpu.SemaphoreType.DMA,
            pltpu.SemaphoreType.DMA,
            pltpu.SemaphoreType.DMA,
            pltpu.SemaphoreType.DMA,
            pltpu.SemaphoreType.DMA,
            pltpu.SemaphoreType.DMA,
        ],
    )


def _embed_body(x_ref, w_ref, b_ref, h_ref):
    h = jnp.dot(x_ref[...], w_ref[...], preferred_element_type=jnp.float32)
    h = h + b_ref[...]
    h_ref[...] = jnp.where(h > 0, h, 0.01 * h)


def _embed(x, w, b):
    return pl.pallas_call(
        _embed_body,
        grid=(NBLK,),
        in_specs=[
            pl.BlockSpec((BLK, F_IN), lambda i: (i, 0)),
            pl.BlockSpec((F_IN, H), lambda i: (0, 0)),
            pl.BlockSpec((1, H), lambda i: (0, 0)),
        ],
        out_specs=pl.BlockSpec((BLK, H), lambda i: (i, 0)),
        out_shape=jax.ShapeDtypeStruct((N, H), jnp.float32),
    )(x, w, b.reshape(1, H))


def _convlin_body(a_ref, h_ref, wrel, wroot, b, c_ref, ssum, ssq, accs, accq):
    i = pl.program_id(0)
    out = (jnp.dot(a_ref[...], wrel[...], preferred_element_type=jnp.float32)
           + jnp.dot(h_ref[...], wroot[...], preferred_element_type=jnp.float32)
           + b[...])
    c_ref[...] = out

    @pl.when(i == 0)
    def _():
        accs[...] = jnp.zeros_like(accs)
        accq[...] = jnp.zeros_like(accq)

    accs[...] += jnp.sum(out, axis=0, keepdims=True)
    accq[...] += jnp.sum(out * out, axis=0, keepdims=True)

    @pl.when(i == NBLK - 1)
    def _():
        ssum[...] = accs[...]
        ssq[...] = accq[...]


def _convlin(a, h, wrel, wroot, b):
    return pl.pallas_call(
        _convlin_body,
        grid=(NBLK,),
        in_specs=[
            pl.BlockSpec((BLK, H), lambda i: (i, 0)),
            pl.BlockSpec((BLK, H), lambda i: (i, 0)),
            pl.BlockSpec((H, H), lambda i: (0, 0)),
            pl.BlockSpec((H, H), lambda i: (0, 0)),
            pl.BlockSpec((1, H), lambda i: (0, 0)),
        ],
        out_specs=[
            pl.BlockSpec((BLK, H), lambda i: (i, 0)),
            pl.BlockSpec((1, H), lambda i: (0, 0)),
            pl.BlockSpec((1, H), lambda i: (0, 0)),
        ],
        out_shape=[
            jax.ShapeDtypeStruct((N, H), jnp.float32),
            jax.ShapeDtypeStruct((1, H), jnp.float32),
            jax.ShapeDtypeStruct((1, H), jnp.float32),
        ],
        scratch_shapes=[
            pltpu.VMEM((1, H), jnp.float32),
            pltpu.VMEM((1, H), jnp.float32),
        ],
    )(a, h, wrel, wroot, b.reshape(1, H))


def _bnapply_body(c_ref, ssum, ssq, gm, bt, id_ref, o_ref):
    mean = ssum[...] * (1.0 / N)
    var = ssq[...] * (1.0 / N) - mean * mean
    scale = gm[...] * lax.rsqrt(var + 1e-5)
    y = (c_ref[...] - mean) * scale + bt[...]
    y = jnp.where(y > 0, y, 0.01 * y)
    o_ref[...] = y + id_ref[...]


def _bnapply(cc, ssum, ssq, gm, bt, iden):
    return pl.pallas_call(
        _bnapply_body,
        grid=(NBLK,),
        in_specs=[
            pl.BlockSpec((BLK, H), lambda i: (i, 0)),
            pl.BlockSpec((1, H), lambda i: (0, 0)),
            pl.BlockSpec((1, H), lambda i: (0, 0)),
            pl.BlockSpec((1, H), lambda i: (0, 0)),
            pl.BlockSpec((1, H), lambda i: (0, 0)),
            pl.BlockSpec((BLK, H), lambda i: (i, 0)),
        ],
        out_specs=pl.BlockSpec((BLK, H), lambda i: (i, 0)),
        out_shape=jax.ShapeDtypeStruct((N, H), jnp.float32),
    )(cc, ssum, ssq, gm.reshape(1, H), bt.reshape(1, H), iden)


def _pool_body(h_ref, bt3, wf1, bf1, wf2, bf2, wf3, bf3, out, accp, accc):
    i = pl.program_id(0)

    @pl.when(i == 0)
    def _():
        accp[...] = jnp.zeros_like(accp)
        accc[...] = jnp.zeros_like(accc)

    bvals = bt3[0, 0, :]
    oh = (bvals[:, None] == lax.broadcasted_iota(jnp.int32, (BLK, G), 1)
          ).astype(jnp.float32)
    accp[...] += lax.dot_general(oh, h_ref[...], (((0,), (0,)), ((), ())),
                                 preferred_element_type=jnp.float32)
    accc[...] += jnp.sum(oh, axis=0).reshape(G, 1)

    @pl.when(i == NBLK - 1)
    def _():
        pooled = accp[...] / jnp.maximum(accc[...], 1.0)
        f1 = jnp.dot(pooled, wf1[...], preferred_element_type=jnp.float32) + bf1[...]
        f1 = jnp.where(f1 > 0, f1, 0.01 * f1)
        f2 = jnp.dot(f1, wf2[...], preferred_element_type=jnp.float32) + bf2[...]
        f2 = jnp.where(f2 > 0, f2, 0.01 * f2)
        out[...] = jnp.dot(f2, wf3[...], preferred_element_type=jnp.float32) + bf3[...]


def _pool_mlp(h, batch, wf1, bf1, wf2, bf2, wf3, bf3):
    bt3 = batch.astype(jnp.int32).reshape(NBLK, 1, BLK)
    return pl.pallas_call(
        _pool_body,
        grid=(NBLK,),
        in_specs=[
            pl.BlockSpec((BLK, H), lambda i: (i, 0)),
            pl.BlockSpec((1, 1, BLK), lambda i: (i, 0, 0)),
            pl.BlockSpec((H, H), lambda i: (0, 0)),
            pl.BlockSpec((1, H), lambda i: (0, 0)),
            pl.BlockSpec((H, H // 2), lambda i: (0, 0)),
            pl.BlockSpec((1, H // 2), lambda i: (0, 0)),
            pl.BlockSpec((H // 2, C_OUT), lambda i: (0, 0)),
            pl.BlockSpec((1, C_OUT), lambda i: (0, 0)),
        ],
        out_specs=pl.BlockSpec((G, C_OUT), lambda i: (0, 0)),
        out_shape=jax.ShapeDtypeStruct((G, C_OUT), jnp.float32),
        scratch_shapes=[
            pltpu.VMEM((G, H), jnp.float32),
            pltpu.VMEM((G, 1), jnp.float32),
        ],
    )(h, bt3, wf1, bf1.reshape(1, H), wf2, bf2.reshape(1, H // 2),
      wf3, bf3.reshape(1, C_OUT))


def kernel(x, edge_index, batch, W_embed, b_embed, W1_rel, b1_rel, W1_root,
           W2_rel, b2_rel, W2_root, bn1_g, bn1_b, bn2_g, bn2_b,
           Wf1, bf1, Wf2, bf2, Wf3, bf3):
    srce = edge_index[0].astype(jnp.int32).reshape(NC * NS, EB)
    dste = edge_index[1].astype(jnp.int32).reshape(NC * NS, EB)
    zeros_in = jnp.zeros((AROWS, 2, 128), jnp.float32)

    comb, counts = _make_bin_call()(srce, dste)
    comb4 = comb.reshape(NC, NC * NS, CAP // 64, 4, CH)
    counts_flat = counts.reshape(-1)
    agg = _make_agg_call()

    def conv_agg(hf):
        o = agg(hf.reshape(N, 2, 128), comb4, counts_flat, zeros_in)
        o = o.reshape(NC * ACC_R, H)
        return jnp.concatenate([o[:SPLIT], o[ACC_R:ACC_R + N - SPLIT]], axis=0)

    hf = _embed(x, W_embed, b_embed)
    a = conv_agg(hf)
    cc, s1, q1 = _convlin(a, hf, W1_rel, W1_root, b1_rel)
    g1 = _bnapply(cc, s1, q1, bn1_g, bn1_b, hf)
    a = conv_agg(g1)
    cc, s2, q2 = _convlin(a, g1, W2_rel, W2_root, b2_rel)
    g2 = _bnapply(cc, s2, q2, bn2_g, bn2_b, hf)
    return _pool_mlp(g2, batch, Wf1, bf1, Wf2, bf2, Wf3, bf3)
